# Initial kernel scaffold; baseline (speedup 1.0000x reference)
import jax
import jax.numpy as jnp
from jax.experimental import pallas as pl


def kernel(texts, char_weights):
    raise NotImplementedError("write your pallas kernel here")


# speedup vs baseline: 3.8013x; 3.8013x over previous
import functools

import jax
import jax.numpy as jnp
from jax import lax
from jax.experimental import pallas as pl
from jax.experimental.pallas import tpu as pltpu
from jax.experimental.pallas import tpu_sc as plsc

DIM = 768
HASH_K = 1315423911 % DIM
L = 16
NC = 2
NS = 16
NW = NC * NS


@functools.lru_cache(maxsize=None)
def _build(batch: int, seq_pad: int, rows_per_chunk: int):
    rows_per_w = batch // NW
    n_chunks = rows_per_w // rows_per_chunk
    mesh = plsc.VectorSubcoreMesh(core_axis_name="c", subcore_axis_name="s")

    @functools.partial(
        pl.kernel,
        out_type=jax.ShapeDtypeStruct((batch, DIM), jnp.float32),
        mesh=mesh,
        compiler_params=pltpu.CompilerParams(needs_layout_passes=False),
        scratch_types=[
            pltpu.VMEM((rows_per_chunk, seq_pad), jnp.int32),
            pltpu.VMEM((rows_per_chunk, seq_pad), jnp.float32),
            pltpu.VMEM((rows_per_chunk, DIM), jnp.float32),
            pltpu.VMEM((DIM,), jnp.float32),
        ],
    )
    def encode(t_hbm, w_hbm, out_hbm, t_v, w_v, o_v, acc):
        wid = lax.axis_index("s") * NC + lax.axis_index("c")
        base = wid * rows_per_w

        def chunk(ci, carry):
            rbase = base + ci * rows_per_chunk
            pltpu.sync_copy(t_hbm.at[pl.ds(rbase, rows_per_chunk)], t_v)
            pltpu.sync_copy(w_hbm.at[pl.ds(rbase, rows_per_chunk)], w_v)

            def row(r, rcarry):
                zero = jnp.zeros((L,), jnp.float32)
                for i in range(DIM // L):
                    acc[pl.ds(i * L, L)] = zero
                for c in range(seq_pad // L):
                    t = t_v[r, pl.ds(c * L, L)]
                    w = w_v[r, pl.ds(c * L, L)]
                    idx = ((t % DIM) * HASH_K) % DIM
                    plsc.addupdate_scatter(acc, [idx], w)
                ss = jnp.zeros((L,), jnp.float32)
                for i in range(DIM // L):
                    v = acc[pl.ds(i * L, L)]
                    ss = ss + v * v
                tot = jnp.broadcast_to(jnp.sum(ss), (L,))
                bits = plsc.bitcast(tot, jnp.int32)
                y = plsc.bitcast(jnp.int32(0x5F3759DF) - (bits >> 1), jnp.float32)
                for _ in range(3):
                    y = y * (1.5 - 0.5 * tot * y * y)
                nrm = jnp.where(tot > 0.0, tot * y, 0.0)
                scale = 1.0 / (nrm + 1e-6)
                for i in range(DIM // L):
                    o_v[r, pl.ds(i * L, L)] = acc[pl.ds(i * L, L)] * scale
                return rcarry

            lax.fori_loop(0, rows_per_chunk, row, 0)
            pltpu.sync_copy(o_v, out_hbm.at[pl.ds(rbase, rows_per_chunk)])
            return carry

        lax.fori_loop(0, n_chunks, chunk, 0)

    return encode


def kernel(texts, char_weights):
    batch, seq = texts.shape
    seq_pad = (seq + L - 1) // L * L
    pad = seq_pad - seq
    if pad:
        texts = jnp.pad(texts, ((0, 0), (0, pad)))
        char_weights = jnp.pad(char_weights, ((0, 0), (0, pad)))
    return _build(batch, seq_pad, 32)(texts, char_weights)


# speedup vs baseline: 4.2165x; 1.1092x over previous
import functools

import jax
import jax.numpy as jnp
from jax import lax
from jax.experimental import pallas as pl
from jax.experimental.pallas import tpu as pltpu
from jax.experimental.pallas import tpu_sc as plsc

DIM = 768
HASH_K = 1315423911 % DIM
L = 16
NC = 2
NS = 16
NW = NC * NS


@functools.lru_cache(maxsize=None)
def _build(batch: int, seq_pad: int, rows_per_chunk: int):
    rows_per_w = batch // NW
    n_chunks = rows_per_w // rows_per_chunk
    n_seq = seq_pad // L
    n_dim = DIM // L
    mesh = plsc.VectorSubcoreMesh(core_axis_name="c", subcore_axis_name="s")

    @functools.partial(
        pl.kernel,
        out_type=jax.ShapeDtypeStruct((batch, DIM), jnp.float32),
        mesh=mesh,
        compiler_params=pltpu.CompilerParams(needs_layout_passes=False),
        scratch_types=[
            pltpu.VMEM((rows_per_chunk, seq_pad), jnp.int32),
            pltpu.VMEM((rows_per_chunk, seq_pad), jnp.float32),
            pltpu.VMEM((rows_per_chunk, seq_pad), jnp.int32),
            pltpu.VMEM((rows_per_chunk, DIM), jnp.float32),
        ],
    )
    def encode(t_hbm, w_hbm, out_hbm, t_v, w_v, idx_s, o_v):
        wid = lax.axis_index("s") * NC + lax.axis_index("c")
        base = wid * rows_per_w
        zero = jnp.zeros((L,), jnp.float32)

        def zrow(r, carry):
            for i in range(n_dim):
                o_v[r, pl.ds(i * L, L)] = zero
            return carry

        lax.fori_loop(0, rows_per_chunk, zrow, 0)

        def chunk(ci, carry):
            rbase = base + ci * rows_per_chunk
            pltpu.sync_copy(t_hbm.at[pl.ds(rbase, rows_per_chunk)], t_v)
            pltpu.sync_copy(w_hbm.at[pl.ds(rbase, rows_per_chunk)], w_v)

            @plsc.parallel_loop(0, rows_per_chunk, 1, unroll=1)
            def row(r):
                rvec = jnp.broadcast_to(r, (L,))
                idxs = []
                ws = []
                for c in range(n_seq):
                    t = t_v[r, pl.ds(c * L, L)]
                    w = w_v[r, pl.ds(c * L, L)]
                    idx = ((t % DIM) * HASH_K) % DIM
                    idx_s[r, pl.ds(c * L, L)] = idx
                    plsc.addupdate_scatter(o_v, [rvec, idx], w)
                    idxs.append(idx)
                    ws.append(w)
                ss = jnp.zeros((L,), jnp.float32)
                vals = []
                for c in range(n_seq):
                    v = plsc.load_gather(o_v, [rvec, idxs[c]])
                    vals.append(v)
                    ss = ss + ws[c] * v
                tot = jnp.broadcast_to(jnp.sum(ss), (L,))
                bits = plsc.bitcast(tot, jnp.int32)
                y = plsc.bitcast(jnp.int32(0x5F3759DF) - (bits >> 1), jnp.float32)
                for _ in range(3):
                    y = y * (1.5 - 0.5 * tot * y * y)
                nrm = jnp.where(tot > 0.0, tot * y, 0.0)
                scale = 1.0 / (nrm + 1e-6)
                for c in range(n_seq):
                    plsc.store_scatter(o_v, [rvec, idxs[c]], vals[c] * scale)

            pltpu.sync_copy(o_v, out_hbm.at[pl.ds(rbase, rows_per_chunk)])

            @plsc.parallel_loop(0, rows_per_chunk, 1, unroll=1)
            def rezero(r):
                rvec = jnp.broadcast_to(r, (L,))
                for c in range(n_seq):
                    idx = idx_s[r, pl.ds(c * L, L)]
                    plsc.store_scatter(o_v, [rvec, idx], zero)

            return carry

        lax.fori_loop(0, n_chunks, chunk, 0)

    return encode


def kernel(texts, char_weights):
    batch, seq = texts.shape
    seq_pad = (seq + L - 1) // L * L
    pad = seq_pad - seq
    if pad:
        texts = jnp.pad(texts, ((0, 0), (0, pad)))
        char_weights = jnp.pad(char_weights, ((0, 0), (0, pad)))
    return _build(batch, seq_pad, 32)(texts, char_weights)
